# baseline (speedup 1.0000x reference)
import functools

import jax
import jax.numpy as jnp
from jax import lax
from jax.experimental import pallas as pl
from jax.experimental.pallas import tpu as pltpu
from jax.experimental.pallas import tpu_sc as plsc

N_NODES = 10000
D = 128
E = 320000
NC, NS, L = 2, 16, 16
NW = NC * NS
EPT = E // NW
K = 80
NCHUNK = EPT // K
NPAD = 10240
RPW = NPAD // NS


def _rn_body(x_ref, rn_ref):
    x = x_ref[...]
    nrm = jnp.sqrt(jnp.sum(x * x, axis=1, keepdims=True))
    rn_ref[...] = 1.0 / jnp.maximum(nrm, 1e-12)


def _sc_agg_body(x_hbm, row_hbm, col_hbm, rn_hbm, z_hbm, aggr_hbm, s_hbm,
                 idxr0, idxr1, idxc0, idxc1, ra0, ra1, rb0, rb1,
                 rnr0, rnr1, rnc0, rnc1,
                 svec, aggr_sh, semi0, semi1, semg0, semg1):
    cid = lax.axis_index("c")
    sid = lax.axis_index("s")
    w = cid * NS + sid
    pltpu.sync_copy(z_hbm.at[pl.ds(sid * RPW, RPW)],
                    aggr_sh.at[pl.ds(sid * RPW, RPW)])
    plsc.subcore_barrier()

    ebase = w * EPT
    idxr = (idxr0, idxr1)
    idxc = (idxc0, idxc1)
    ra = (ra0, ra1)
    rb = (rb0, rb1)
    rnr_g = (rnr0, rnr1)
    rnc_g = (rnc0, rnc1)
    semi = (semi0, semi1)
    semg = (semg0, semg1)

    def issue_idx(c, p):
        eb = ebase + c * K
        pltpu.async_copy(row_hbm.at[pl.ds(eb, K)], idxr[p], semi[p])
        pltpu.async_copy(col_hbm.at[pl.ds(eb, K)], idxc[p], semi[p])

    def wait_idx(p):
        pltpu.make_async_copy(row_hbm.at[pl.ds(0, K)], idxr[p], semi[p]).wait()
        pltpu.make_async_copy(col_hbm.at[pl.ds(0, K)], idxc[p], semi[p]).wait()

    def issue_gather(p):
        pltpu.async_copy(x_hbm.at[idxr[p]], ra[p], semg[p])
        pltpu.async_copy(x_hbm.at[idxc[p]], rb[p], semg[p])
        pltpu.async_copy(rn_hbm.at[idxr[p]], rnr_g[p], semg[p])
        pltpu.async_copy(rn_hbm.at[idxc[p]], rnc_g[p], semg[p])

    def wait_gather(p):
        pltpu.make_async_copy(x_hbm.at[idxr[p]], ra[p], semg[p]).wait()
        pltpu.make_async_copy(x_hbm.at[idxc[p]], rb[p], semg[p]).wait()
        pltpu.make_async_copy(rn_hbm.at[idxr[p]], rnr_g[p], semg[p]).wait()
        pltpu.make_async_copy(rn_hbm.at[idxc[p]], rnc_g[p], semg[p]).wait()

    def chunk_compute(p, s_acc):
        rows_a, rows_b = ra[p], rb[p]
        idxr_v = idxr[p]
        rnr_v, rnc_v = rnr_g[p], rnc_g[p]

        def group_body(g, s_in):
            rid = g * L + lax.iota(jnp.int32, L)
            rnr = rnr_v[pl.ds(g * L, L)]
            rnc = rnc_v[pl.ds(g * L, L)]
            def dot_body(f0, acc):
                ps = []
                for df in range(8):
                    fv = f0 + jnp.full((L,), df, jnp.int32)
                    a = plsc.load_gather(rows_a, [rid, fv])
                    bv = plsc.load_gather(rows_b, [rid, fv])
                    ps.append(a * bv)
                t = (((ps[0] + ps[1]) + (ps[2] + ps[3]))
                     + ((ps[4] + ps[5]) + (ps[6] + ps[7])))
                return acc + t

            acc = plsc.parallel_loop(
                0, D, step=8, carry=jnp.zeros((L,), jnp.float32))(dot_body)
            coeff = jnp.exp(acc * rnr * rnc - 1.0)

            def scale_body(f0):
                for df in range(8):
                    fv = f0 + jnp.full((L,), df, jnp.int32)
                    bv = plsc.load_gather(rows_b, [rid, fv])
                    plsc.store_scatter(rows_b, [rid, fv], bv * coeff)

            plsc.parallel_loop(0, D, step=8)(scale_body)
            return s_in + coeff

        s_acc = lax.fori_loop(0, K // L, group_body, s_acc)
        pltpu.sync_copy(rows_b, aggr_sh.at[idxr_v], add=True)
        return s_acc

    issue_idx(0, 0)
    issue_idx(1, 1)
    wait_idx(0)
    issue_gather(0)

    def outer(t, s_acc):
        c0 = t * 2
        wait_gather(0)
        wait_idx(1)
        issue_gather(1)
        s_acc = chunk_compute(0, s_acc)

        @pl.when(c0 + 2 < NCHUNK)
        def _():
            issue_idx(c0 + 2, 0)

        wait_gather(1)

        @pl.when(c0 + 2 < NCHUNK)
        def _():
            wait_idx(0)
            issue_gather(0)

        s_acc = chunk_compute(1, s_acc)

        @pl.when(c0 + 3 < NCHUNK)
        def _():
            issue_idx(c0 + 3, 1)

        return s_acc

    s_acc = lax.fori_loop(0, (NCHUNK - 1) // 2, outer,
                          jnp.zeros((L,), jnp.float32))
    wait_gather(0)
    s_acc = chunk_compute(0, s_acc)
    svec[...] = s_acc
    pltpu.sync_copy(svec, s_hbm.at[w])
    plsc.subcore_barrier()
    pltpu.sync_copy(aggr_sh.at[pl.ds(sid * RPW, RPW)],
                    aggr_hbm.at[pl.ds(cid * NPAD + sid * RPW, RPW)])


_sc_agg = functools.partial(
    pl.kernel,
    out_type=[jax.ShapeDtypeStruct((NC * NPAD, D), jnp.float32),
              jax.ShapeDtypeStruct((NW, L), jnp.float32)],
    mesh=plsc.VectorSubcoreMesh(core_axis_name="c", subcore_axis_name="s"),
    compiler_params=pltpu.CompilerParams(needs_layout_passes=False),
    scratch_types=[
        pltpu.VMEM((K,), jnp.int32),
        pltpu.VMEM((K,), jnp.int32),
        pltpu.VMEM((K,), jnp.int32),
        pltpu.VMEM((K,), jnp.int32),
        pltpu.VMEM((K, D), jnp.float32),
        pltpu.VMEM((K, D), jnp.float32),
        pltpu.VMEM((K, D), jnp.float32),
        pltpu.VMEM((K, D), jnp.float32),
        pltpu.VMEM((K,), jnp.float32),
        pltpu.VMEM((K,), jnp.float32),
        pltpu.VMEM((K,), jnp.float32),
        pltpu.VMEM((K,), jnp.float32),
        pltpu.VMEM((L,), jnp.float32),
        pltpu.VMEM_SHARED((NPAD, D), jnp.float32),
        pltpu.SemaphoreType.DMA,
        pltpu.SemaphoreType.DMA,
        pltpu.SemaphoreType.DMA,
        pltpu.SemaphoreType.DMA,
    ],
)(_sc_agg_body)


def _post_body(a_ref, s_ref, x_ref, w_ref, b_ref, o_ref):
    a = a_ref[:N_NODES, :] + a_ref[NPAD:NPAD + N_NODES, :]
    s_total = jnp.sum(s_ref[...])
    m = lax.dot_general(a, w_ref[...], (((1,), (1,)), ((), ())),
                        preferred_element_type=jnp.float32)
    o_ref[...] = m * (1.0 / s_total) + b_ref[...] + x_ref[...]


def kernel(x, edge_index, W, b):
    row = edge_index[0]
    col = edge_index[1]
    rn = pl.pallas_call(
        _rn_body,
        out_shape=jax.ShapeDtypeStruct((N_NODES, 1), jnp.float32),
    )(x)
    zeros = jnp.zeros((NPAD, D), jnp.float32)
    aggr2, s_parts = _sc_agg(x, row, col, rn.reshape(N_NODES), zeros)
    out = pl.pallas_call(
        _post_body,
        out_shape=jax.ShapeDtypeStruct((N_NODES, D), jnp.float32),
    )(aggr2, s_parts, x, W, b.reshape(1, D))
    return out


# speedup vs baseline: 6.7483x; 6.7483x over previous
import functools

import jax
import jax.numpy as jnp
from jax import lax
from jax.experimental import pallas as pl
from jax.experimental.pallas import tpu as pltpu
from jax.experimental.pallas import tpu_sc as plsc

N_NODES = 10000
D = 128
E = 320000
NC, NS, L = 2, 16, 16
NW = NC * NS
EPT = E // NW
K = 80
NCHUNK = EPT // K
NPAD = 10240
RPW = NPAD // NS


def _rn_body(x_ref, rn_ref):
    x = x_ref[...]
    nrm = jnp.sqrt(jnp.sum(x * x, axis=1, keepdims=True))
    rn_ref[...] = 1.0 / jnp.maximum(nrm, 1e-12)


def _sc_agg_body(x_hbm, row_hbm, col_hbm, rn_hbm, z_hbm, aggr_hbm, s_hbm,
                 idxr0, idxr1, idxc0, idxc1, ra0, ra1, rb0, rb1,
                 rnr0, rnr1, rnc0, rnc1,
                 svec, aggr_sh, semi0, semi1, semg0, semg1):
    cid = lax.axis_index("c")
    sid = lax.axis_index("s")
    w = cid * NS + sid
    pltpu.sync_copy(z_hbm.at[pl.ds(sid * RPW, RPW)],
                    aggr_sh.at[pl.ds(sid * RPW, RPW)])
    plsc.subcore_barrier()

    ebase = w * EPT
    idxr = (idxr0, idxr1)
    idxc = (idxc0, idxc1)
    ra = (ra0, ra1)
    rb = (rb0, rb1)
    rnr_g = (rnr0, rnr1)
    rnc_g = (rnc0, rnc1)
    semi = (semi0, semi1)
    semg = (semg0, semg1)

    def issue_idx(c, p):
        eb = ebase + c * K
        pltpu.async_copy(row_hbm.at[pl.ds(eb, K)], idxr[p], semi[p])
        pltpu.async_copy(col_hbm.at[pl.ds(eb, K)], idxc[p], semi[p])

    def wait_idx(p):
        pltpu.make_async_copy(row_hbm.at[pl.ds(0, K)], idxr[p], semi[p]).wait()
        pltpu.make_async_copy(col_hbm.at[pl.ds(0, K)], idxc[p], semi[p]).wait()

    def issue_gather(p):
        pltpu.async_copy(x_hbm.at[idxr[p]], ra[p], semg[p])
        pltpu.async_copy(x_hbm.at[idxc[p]], rb[p], semg[p])
        pltpu.async_copy(rn_hbm.at[idxr[p]], rnr_g[p], semg[p])
        pltpu.async_copy(rn_hbm.at[idxc[p]], rnc_g[p], semg[p])

    def wait_gather(p):
        pltpu.make_async_copy(x_hbm.at[idxr[p]], ra[p], semg[p]).wait()
        pltpu.make_async_copy(x_hbm.at[idxc[p]], rb[p], semg[p]).wait()
        pltpu.make_async_copy(rn_hbm.at[idxr[p]], rnr_g[p], semg[p]).wait()
        pltpu.make_async_copy(rn_hbm.at[idxc[p]], rnc_g[p], semg[p]).wait()

    lane0 = lax.iota(jnp.int32, L) == 0
    zerov = jnp.zeros((L,), jnp.float32)

    def chunk_compute(p, s_acc):
        rows_a, rows_b = ra[p], rb[p]
        idxr_v = idxr[p]
        rnr_v, rnc_v = rnr_g[p], rnc_g[p]

        def edge_body(j, s_in):
            jf = jnp.full((L,), j, jnp.int32)
            av = [rows_a[j, pl.ds(k * L, L)] for k in range(D // L)]
            bv = [rows_b[j, pl.ds(k * L, L)] for k in range(D // L)]
            ps = [a * b for a, b in zip(av, bv)]
            t = (((ps[0] + ps[1]) + (ps[2] + ps[3]))
                 + ((ps[4] + ps[5]) + (ps[6] + ps[7])))
            dot_s = jnp.sum(t)
            rnrv = plsc.load_gather(rnr_v, [jf])
            rncv = plsc.load_gather(rnc_v, [jf])
            coeffv = jnp.exp(jnp.full((L,), dot_s) * rnrv * rncv - 1.0)
            for k in range(D // L):
                rows_b[j, pl.ds(k * L, L)] = bv[k] * coeffv
            return s_in + jnp.where(lane0, coeffv, zerov)

        s_acc = plsc.parallel_loop(0, K, carry=s_acc)(edge_body)
        pltpu.sync_copy(rows_b, aggr_sh.at[idxr_v], add=True)
        return s_acc

    issue_idx(0, 0)
    issue_idx(1, 1)
    wait_idx(0)
    issue_gather(0)

    def outer(t, s_acc):
        c0 = t * 2
        wait_gather(0)
        wait_idx(1)
        issue_gather(1)
        s_acc = chunk_compute(0, s_acc)

        @pl.when(c0 + 2 < NCHUNK)
        def _():
            issue_idx(c0 + 2, 0)

        wait_gather(1)

        @pl.when(c0 + 2 < NCHUNK)
        def _():
            wait_idx(0)
            issue_gather(0)

        s_acc = chunk_compute(1, s_acc)

        @pl.when(c0 + 3 < NCHUNK)
        def _():
            issue_idx(c0 + 3, 1)

        return s_acc

    s_acc = lax.fori_loop(0, (NCHUNK - 1) // 2, outer,
                          jnp.zeros((L,), jnp.float32))
    wait_gather(0)
    s_acc = chunk_compute(0, s_acc)
    svec[...] = s_acc
    pltpu.sync_copy(svec, s_hbm.at[w])
    plsc.subcore_barrier()
    pltpu.sync_copy(aggr_sh.at[pl.ds(sid * RPW, RPW)],
                    aggr_hbm.at[pl.ds(cid * NPAD + sid * RPW, RPW)])


_sc_agg = functools.partial(
    pl.kernel,
    out_type=[jax.ShapeDtypeStruct((NC * NPAD, D), jnp.float32),
              jax.ShapeDtypeStruct((NW, L), jnp.float32)],
    mesh=plsc.VectorSubcoreMesh(core_axis_name="c", subcore_axis_name="s"),
    compiler_params=pltpu.CompilerParams(needs_layout_passes=False),
    scratch_types=[
        pltpu.VMEM((K,), jnp.int32),
        pltpu.VMEM((K,), jnp.int32),
        pltpu.VMEM((K,), jnp.int32),
        pltpu.VMEM((K,), jnp.int32),
        pltpu.VMEM((K, D), jnp.float32),
        pltpu.VMEM((K, D), jnp.float32),
        pltpu.VMEM((K, D), jnp.float32),
        pltpu.VMEM((K, D), jnp.float32),
        pltpu.VMEM((K,), jnp.float32),
        pltpu.VMEM((K,), jnp.float32),
        pltpu.VMEM((K,), jnp.float32),
        pltpu.VMEM((K,), jnp.float32),
        pltpu.VMEM((L,), jnp.float32),
        pltpu.VMEM_SHARED((NPAD, D), jnp.float32),
        pltpu.SemaphoreType.DMA,
        pltpu.SemaphoreType.DMA,
        pltpu.SemaphoreType.DMA,
        pltpu.SemaphoreType.DMA,
    ],
)(_sc_agg_body)


def _post_body(a_ref, s_ref, x_ref, w_ref, b_ref, o_ref):
    a = a_ref[:N_NODES, :] + a_ref[NPAD:NPAD + N_NODES, :]
    s_total = jnp.sum(s_ref[...])
    m = lax.dot_general(a, w_ref[...], (((1,), (1,)), ((), ())),
                        preferred_element_type=jnp.float32)
    o_ref[...] = m * (1.0 / s_total) + b_ref[...] + x_ref[...]


def kernel(x, edge_index, W, b):
    row = edge_index[0]
    col = edge_index[1]
    rn = pl.pallas_call(
        _rn_body,
        out_shape=jax.ShapeDtypeStruct((N_NODES, 1), jnp.float32),
    )(x)
    zeros = jnp.zeros((NPAD, D), jnp.float32)
    aggr2, s_parts = _sc_agg(x, row, col, rn.reshape(N_NODES), zeros)
    out = pl.pallas_call(
        _post_body,
        out_shape=jax.ShapeDtypeStruct((N_NODES, D), jnp.float32),
    )(aggr2, s_parts, x, W, b.reshape(1, D))
    return out


# speedup vs baseline: 6.7649x; 1.0025x over previous
import functools

import jax
import jax.numpy as jnp
from jax import lax
from jax.experimental import pallas as pl
from jax.experimental.pallas import tpu as pltpu
from jax.experimental.pallas import tpu_sc as plsc

N_NODES = 10000
D = 128
E = 320000
NC, NS, L = 2, 16, 16
NW = NC * NS
EPT = E // NW
K = 80
NCHUNK = EPT // K
NPAD = 10240
RPW = NPAD // NS


def _rn_body(x_ref, rn_ref):
    x = x_ref[...]
    nrm = jnp.sqrt(jnp.sum(x * x, axis=1, keepdims=True))
    rn_ref[...] = 1.0 / jnp.maximum(nrm, 1e-12)


def _sc_agg_body(x_hbm, row_hbm, col_hbm, rn_hbm, z_hbm, aggr_hbm, s_hbm,
                 idxr0, idxr1, idxc0, idxc1, ra0, ra1, rb0, rb1,
                 rnr0, rnr1, rnc0, rnc1, sidx0, sidx1,
                 svec, aggr_sh, semi0, semi1, semg0, semg1, sems0, sems1):
    cid = lax.axis_index("c")
    sid = lax.axis_index("s")
    w = cid * NS + sid
    pltpu.sync_copy(z_hbm.at[pl.ds(sid * RPW, RPW)],
                    aggr_sh.at[pl.ds(sid * RPW, RPW)])
    plsc.subcore_barrier()

    ebase = w * EPT
    idxr = (idxr0, idxr1)
    idxc = (idxc0, idxc1)
    ra = (ra0, ra1)
    rb = (rb0, rb1)
    rnr_g = (rnr0, rnr1)
    rnc_g = (rnc0, rnc1)
    sidx = (sidx0, sidx1)
    semi = (semi0, semi1)
    semg = (semg0, semg1)
    sems = (sems0, sems1)

    def issue_idx(c, p):
        eb = ebase + c * K
        pltpu.async_copy(row_hbm.at[pl.ds(eb, K)], idxr[p], semi[p])
        pltpu.async_copy(col_hbm.at[pl.ds(eb, K)], idxc[p], semi[p])

    def wait_idx(p):
        pltpu.make_async_copy(row_hbm.at[pl.ds(0, K)], idxr[p], semi[p]).wait()
        pltpu.make_async_copy(col_hbm.at[pl.ds(0, K)], idxc[p], semi[p]).wait()

    def issue_gather(p):
        pltpu.async_copy(x_hbm.at[idxr[p]], ra[p], semg[p])
        pltpu.async_copy(x_hbm.at[idxc[p]], rb[p], semg[p])
        pltpu.async_copy(rn_hbm.at[idxr[p]], rnr_g[p], semg[p])
        pltpu.async_copy(rn_hbm.at[idxc[p]], rnc_g[p], semg[p])

    def wait_gather(p):
        pltpu.make_async_copy(x_hbm.at[idxr[p]], ra[p], semg[p]).wait()
        pltpu.make_async_copy(x_hbm.at[idxc[p]], rb[p], semg[p]).wait()
        pltpu.make_async_copy(rn_hbm.at[idxr[p]], rnr_g[p], semg[p]).wait()
        pltpu.make_async_copy(rn_hbm.at[idxc[p]], rnc_g[p], semg[p]).wait()

    lane0 = lax.iota(jnp.int32, L) == 0
    zerov = jnp.zeros((L,), jnp.float32)

    def chunk_compute(p, s_acc):
        rows_a, rows_b = ra[p], rb[p]
        idxr_v = idxr[p]
        rnr_v, rnc_v = rnr_g[p], rnc_g[p]

        def edge_body(j, s_in):
            jf = jnp.full((L,), j, jnp.int32)
            av = [rows_a[j, pl.ds(k * L, L)] for k in range(D // L)]
            bv = [rows_b[j, pl.ds(k * L, L)] for k in range(D // L)]
            ps = [a * b for a, b in zip(av, bv)]
            t = (((ps[0] + ps[1]) + (ps[2] + ps[3]))
                 + ((ps[4] + ps[5]) + (ps[6] + ps[7])))
            dot_s = jnp.sum(t)
            rnrv = plsc.load_gather(rnr_v, [jf])
            rncv = plsc.load_gather(rnc_v, [jf])
            coeffv = jnp.exp(jnp.full((L,), dot_s) * rnrv * rncv - 1.0)
            for k in range(D // L):
                rows_b[j, pl.ds(k * L, L)] = bv[k] * coeffv
            return s_in + jnp.where(lane0, coeffv, zerov)

        s_acc = plsc.parallel_loop(0, K, carry=s_acc)(edge_body)
        for q in range(K // L):
            sidx[p][pl.ds(q * L, L)] = idxr_v[pl.ds(q * L, L)]
        pltpu.async_copy(rows_b, aggr_sh.at[sidx[p]], sems[p], add=True)
        return s_acc

    def wait_scatter(p):
        pltpu.make_async_copy(rb[p], aggr_sh.at[sidx[p]], sems[p]).wait()

    issue_idx(0, 0)
    issue_idx(1, 1)
    wait_idx(0)
    issue_gather(0)

    def outer(t, s_acc):
        c0 = t * 2
        wait_gather(0)

        @pl.when(c0 > 0)
        def _():
            wait_scatter(1)

        wait_idx(1)
        issue_gather(1)
        s_acc = chunk_compute(0, s_acc)

        @pl.when(c0 + 2 < NCHUNK)
        def _():
            issue_idx(c0 + 2, 0)

        wait_gather(1)

        @pl.when(c0 + 2 < NCHUNK)
        def _():
            wait_scatter(0)
            wait_idx(0)
            issue_gather(0)

        s_acc = chunk_compute(1, s_acc)

        @pl.when(c0 + 3 < NCHUNK)
        def _():
            issue_idx(c0 + 3, 1)

        return s_acc

    s_acc = lax.fori_loop(0, (NCHUNK - 1) // 2, outer,
                          jnp.zeros((L,), jnp.float32))
    wait_gather(0)
    s_acc = chunk_compute(0, s_acc)
    wait_scatter(0)
    wait_scatter(1)
    svec[...] = s_acc
    pltpu.sync_copy(svec, s_hbm.at[w])
    plsc.subcore_barrier()
    pltpu.sync_copy(aggr_sh.at[pl.ds(sid * RPW, RPW)],
                    aggr_hbm.at[pl.ds(cid * NPAD + sid * RPW, RPW)])


_sc_agg = functools.partial(
    pl.kernel,
    out_type=[jax.ShapeDtypeStruct((NC * NPAD, D), jnp.float32),
              jax.ShapeDtypeStruct((NW, L), jnp.float32)],
    mesh=plsc.VectorSubcoreMesh(core_axis_name="c", subcore_axis_name="s"),
    compiler_params=pltpu.CompilerParams(needs_layout_passes=False),
    scratch_types=[
        pltpu.VMEM((K,), jnp.int32),
        pltpu.VMEM((K,), jnp.int32),
        pltpu.VMEM((K,), jnp.int32),
        pltpu.VMEM((K,), jnp.int32),
        pltpu.VMEM((K, D), jnp.float32),
        pltpu.VMEM((K, D), jnp.float32),
        pltpu.VMEM((K, D), jnp.float32),
        pltpu.VMEM((K, D), jnp.float32),
        pltpu.VMEM((K,), jnp.float32),
        pltpu.VMEM((K,), jnp.float32),
        pltpu.VMEM((K,), jnp.float32),
        pltpu.VMEM((K,), jnp.float32),
        pltpu.VMEM((K,), jnp.int32),
        pltpu.VMEM((K,), jnp.int32),
        pltpu.VMEM((L,), jnp.float32),
        pltpu.VMEM_SHARED((NPAD, D), jnp.float32),
        pltpu.SemaphoreType.DMA,
        pltpu.SemaphoreType.DMA,
        pltpu.SemaphoreType.DMA,
        pltpu.SemaphoreType.DMA,
        pltpu.SemaphoreType.DMA,
        pltpu.SemaphoreType.DMA,
    ],
)(_sc_agg_body)


def _post_body(a_ref, s_ref, x_ref, w_ref, b_ref, o_ref):
    a = a_ref[:N_NODES, :] + a_ref[NPAD:NPAD + N_NODES, :]
    s_total = jnp.sum(s_ref[...])
    m = lax.dot_general(a, w_ref[...], (((1,), (1,)), ((), ())),
                        preferred_element_type=jnp.float32)
    o_ref[...] = m * (1.0 / s_total) + b_ref[...] + x_ref[...]


def kernel(x, edge_index, W, b):
    row = edge_index[0]
    col = edge_index[1]
    rn = pl.pallas_call(
        _rn_body,
        out_shape=jax.ShapeDtypeStruct((N_NODES, 1), jnp.float32),
    )(x)
    zeros = jnp.zeros((NPAD, D), jnp.float32)
    aggr2, s_parts = _sc_agg(x, row, col, rn.reshape(N_NODES), zeros)
    out = pl.pallas_call(
        _post_body,
        out_shape=jax.ShapeDtypeStruct((N_NODES, D), jnp.float32),
    )(aggr2, s_parts, x, W, b.reshape(1, D))
    return out


# speedup vs baseline: 6.8730x; 1.0160x over previous
import functools

import jax
import jax.numpy as jnp
from jax import lax
from jax.experimental import pallas as pl
from jax.experimental.pallas import tpu as pltpu
from jax.experimental.pallas import tpu_sc as plsc

N_NODES = 10000
D = 128
E = 320000
NC, NS, L = 2, 16, 16
NW = NC * NS
EPT = E // NW
K = 80
NCHUNK = EPT // K
NPAD = 10240
RPW = NPAD // NS


def _rn_body(x_ref, rn_ref):
    x = x_ref[...]
    nrm = jnp.sqrt(jnp.sum(x * x, axis=1, keepdims=True))
    rn_ref[...] = 1.0 / jnp.maximum(nrm, 1e-12)


def _sc_agg_body(x_hbm, row_hbm, col_hbm, rn_hbm, aggr_hbm, s_hbm,
                 idxr0, idxr1, idxc0, idxc1, ra0, ra1, rb0, rb1,
                 rnr0, rnr1, rnc0, rnc1, sidx0, sidx1,
                 svec, aggr_sh, semi0, semi1, semg0, semg1, sems0, sems1):
    cid = lax.axis_index("c")
    sid = lax.axis_index("s")
    w = cid * NS + sid

    zv = jnp.zeros((L,), jnp.float32)

    def zrow(j, _):
        for k in range(D // L):
            rb0[j, pl.ds(k * L, L)] = zv
        return _

    lax.fori_loop(0, K, zrow, jnp.int32(0))
    for q in range(RPW // K):
        pltpu.sync_copy(rb0, aggr_sh.at[pl.ds(sid * RPW + q * K, K)])
    plsc.subcore_barrier()

    ebase = w * EPT
    idxr = (idxr0, idxr1)
    idxc = (idxc0, idxc1)
    ra = (ra0, ra1)
    rb = (rb0, rb1)
    rnr_g = (rnr0, rnr1)
    rnc_g = (rnc0, rnc1)
    sidx = (sidx0, sidx1)
    semi = (semi0, semi1)
    semg = (semg0, semg1)
    sems = (sems0, sems1)

    def issue_idx(c, p):
        eb = ebase + c * K
        pltpu.async_copy(row_hbm.at[pl.ds(eb, K)], idxr[p], semi[p])
        pltpu.async_copy(col_hbm.at[pl.ds(eb, K)], idxc[p], semi[p])

    def wait_idx(p):
        pltpu.make_async_copy(row_hbm.at[pl.ds(0, K)], idxr[p], semi[p]).wait()
        pltpu.make_async_copy(col_hbm.at[pl.ds(0, K)], idxc[p], semi[p]).wait()

    def issue_gather(p):
        pltpu.async_copy(x_hbm.at[idxr[p]], ra[p], semg[p])
        pltpu.async_copy(x_hbm.at[idxc[p]], rb[p], semg[p])
        pltpu.async_copy(rn_hbm.at[idxr[p]], rnr_g[p], semg[p])
        pltpu.async_copy(rn_hbm.at[idxc[p]], rnc_g[p], semg[p])

    def wait_gather(p):
        pltpu.make_async_copy(x_hbm.at[idxr[p]], ra[p], semg[p]).wait()
        pltpu.make_async_copy(x_hbm.at[idxc[p]], rb[p], semg[p]).wait()
        pltpu.make_async_copy(rn_hbm.at[idxr[p]], rnr_g[p], semg[p]).wait()
        pltpu.make_async_copy(rn_hbm.at[idxc[p]], rnc_g[p], semg[p]).wait()

    lane0 = lax.iota(jnp.int32, L) == 0
    zerov = jnp.zeros((L,), jnp.float32)

    def chunk_compute(p, s_acc):
        rows_a, rows_b = ra[p], rb[p]
        idxr_v = idxr[p]
        rnr_v, rnc_v = rnr_g[p], rnc_g[p]

        def edge_body(j, s_in):
            jf = jnp.full((L,), j, jnp.int32)
            av = [rows_a[j, pl.ds(k * L, L)] for k in range(D // L)]
            bv = [rows_b[j, pl.ds(k * L, L)] for k in range(D // L)]
            ps = [a * b for a, b in zip(av, bv)]
            t = (((ps[0] + ps[1]) + (ps[2] + ps[3]))
                 + ((ps[4] + ps[5]) + (ps[6] + ps[7])))
            dot_s = jnp.sum(t)
            rnrv = plsc.load_gather(rnr_v, [jf])
            rncv = plsc.load_gather(rnc_v, [jf])
            coeffv = jnp.exp(jnp.full((L,), dot_s) * rnrv * rncv - 1.0)
            for k in range(D // L):
                rows_b[j, pl.ds(k * L, L)] = bv[k] * coeffv
            return s_in + jnp.where(lane0, coeffv, zerov)

        s_acc = plsc.parallel_loop(0, K, carry=s_acc)(edge_body)
        for q in range(K // L):
            sidx[p][pl.ds(q * L, L)] = idxr_v[pl.ds(q * L, L)]
        pltpu.async_copy(rows_b, aggr_sh.at[sidx[p]], sems[p], add=True)
        return s_acc

    def wait_scatter(p):
        pltpu.make_async_copy(rb[p], aggr_sh.at[sidx[p]], sems[p]).wait()

    issue_idx(0, 0)
    issue_idx(1, 1)
    wait_idx(0)
    issue_gather(0)

    def outer(t, s_acc):
        c0 = t * 2
        wait_gather(0)

        @pl.when(c0 > 0)
        def _():
            wait_scatter(1)

        wait_idx(1)
        issue_gather(1)
        s_acc = chunk_compute(0, s_acc)

        @pl.when(c0 + 2 < NCHUNK)
        def _():
            issue_idx(c0 + 2, 0)

        wait_gather(1)

        @pl.when(c0 + 2 < NCHUNK)
        def _():
            wait_scatter(0)
            wait_idx(0)
            issue_gather(0)

        s_acc = chunk_compute(1, s_acc)

        @pl.when(c0 + 3 < NCHUNK)
        def _():
            issue_idx(c0 + 3, 1)

        return s_acc

    s_acc = lax.fori_loop(0, (NCHUNK - 1) // 2, outer,
                          jnp.zeros((L,), jnp.float32))
    wait_gather(0)
    s_acc = chunk_compute(0, s_acc)
    wait_scatter(0)
    wait_scatter(1)
    svec[...] = s_acc
    pltpu.sync_copy(svec, s_hbm.at[w])
    plsc.subcore_barrier()
    pltpu.sync_copy(aggr_sh.at[pl.ds(sid * RPW, RPW)],
                    aggr_hbm.at[pl.ds(cid * NPAD + sid * RPW, RPW)])


_sc_agg = functools.partial(
    pl.kernel,
    out_type=[jax.ShapeDtypeStruct((NC * NPAD, D), jnp.float32),
              jax.ShapeDtypeStruct((NW, L), jnp.float32)],
    mesh=plsc.VectorSubcoreMesh(core_axis_name="c", subcore_axis_name="s"),
    compiler_params=pltpu.CompilerParams(needs_layout_passes=False),
    scratch_types=[
        pltpu.VMEM((K,), jnp.int32),
        pltpu.VMEM((K,), jnp.int32),
        pltpu.VMEM((K,), jnp.int32),
        pltpu.VMEM((K,), jnp.int32),
        pltpu.VMEM((K, D), jnp.float32),
        pltpu.VMEM((K, D), jnp.float32),
        pltpu.VMEM((K, D), jnp.float32),
        pltpu.VMEM((K, D), jnp.float32),
        pltpu.VMEM((K,), jnp.float32),
        pltpu.VMEM((K,), jnp.float32),
        pltpu.VMEM((K,), jnp.float32),
        pltpu.VMEM((K,), jnp.float32),
        pltpu.VMEM((K,), jnp.int32),
        pltpu.VMEM((K,), jnp.int32),
        pltpu.VMEM((L,), jnp.float32),
        pltpu.VMEM_SHARED((NPAD, D), jnp.float32),
        pltpu.SemaphoreType.DMA,
        pltpu.SemaphoreType.DMA,
        pltpu.SemaphoreType.DMA,
        pltpu.SemaphoreType.DMA,
        pltpu.SemaphoreType.DMA,
        pltpu.SemaphoreType.DMA,
    ],
)(_sc_agg_body)


def _post_body(a_ref, s_ref, x_ref, w_ref, b_ref, o_ref):
    a = a_ref[:N_NODES, :] + a_ref[NPAD:NPAD + N_NODES, :]
    s_total = jnp.sum(s_ref[...])
    m = lax.dot_general(a, w_ref[...], (((1,), (1,)), ((), ())),
                        preferred_element_type=jnp.float32)
    o_ref[...] = m * (1.0 / s_total) + b_ref[...] + x_ref[...]


def kernel(x, edge_index, W, b):
    row = edge_index[0]
    col = edge_index[1]
    rn = pl.pallas_call(
        _rn_body,
        out_shape=jax.ShapeDtypeStruct((N_NODES, 1), jnp.float32),
    )(x)
    aggr2, s_parts = _sc_agg(x, row, col, rn.reshape(N_NODES))
    out = pl.pallas_call(
        _post_body,
        out_shape=jax.ShapeDtypeStruct((N_NODES, D), jnp.float32),
    )(aggr2, s_parts, x, W, b.reshape(1, D))
    return out
